# Initial kernel scaffold; baseline (speedup 1.0000x reference)
import jax
import jax.numpy as jnp
from jax.experimental import pallas as pl


def kernel(input_ids, attention_mask, W):
    raise NotImplementedError("write your pallas kernel here")


# speedup vs baseline: 1.1849x; 1.1849x over previous
import functools

import jax
import jax.numpy as jnp
from jax import lax
from jax.experimental import pallas as pl
from jax.experimental.pallas import tpu as pltpu
from jax.experimental.pallas import tpu_sc as plsc

OUT_DIM = 128
B_TOTAL = 1024 * 300
NUM_WORKERS = 32
BPW = B_TOTAL // NUM_WORKERS
CHUNK = 320
NCHUNK = BPW // CHUNK


def _make_gather():
    mesh = plsc.VectorSubcoreMesh(core_axis_name="c", subcore_axis_name="s")

    @functools.partial(
        pl.kernel,
        mesh=mesh,
        out_type=jax.ShapeDtypeStruct((B_TOTAL, OUT_DIM), jnp.float32),
        scratch_types=[
            pltpu.VMEM((BPW,), jnp.int32),
            pltpu.VMEM((CHUNK, OUT_DIM), jnp.float32),
            pltpu.SemaphoreType.DMA,
        ],
    )
    def gather_kernel(idx_hbm, table_hbm, out_hbm, idx_v, rows_v, sem):
        wid = lax.axis_index("s") * 2 + lax.axis_index("c")
        base = wid * BPW
        pltpu.sync_copy(idx_hbm.at[pl.ds(base, BPW)], idx_v)

        def body(c, carry):
            off = c * CHUNK
            pltpu.async_copy(
                table_hbm.at[idx_v.at[pl.ds(off, CHUNK)]], rows_v, sem
            ).wait()
            pltpu.sync_copy(rows_v, out_hbm.at[pl.ds(base + off, CHUNK)])
            return carry

        lax.fori_loop(0, NCHUNK, body, 0)

    return gather_kernel


_gather = _make_gather()


def kernel(input_ids, attention_mask, W):
    ids_flat = input_ids.reshape(-1)
    embeds = _gather(ids_flat, W)
    embeds = embeds.reshape(input_ids.shape[0], input_ids.shape[1], OUT_DIM)
    return (embeds, embeds, attention_mask)


# speedup vs baseline: 1.2160x; 1.0262x over previous
import functools

import jax
import jax.numpy as jnp
from jax import lax
from jax.experimental import pallas as pl
from jax.experimental.pallas import tpu as pltpu
from jax.experimental.pallas import tpu_sc as plsc

OUT_DIM = 128
B_TOTAL = 1024 * 300
NUM_WORKERS = 32
BPW = B_TOTAL // NUM_WORKERS
CHUNK = 200
NBUF = 4
NCHUNK = BPW // CHUNK


def _make_gather():
    mesh = plsc.VectorSubcoreMesh(core_axis_name="c", subcore_axis_name="s")

    @functools.partial(
        pl.kernel,
        mesh=mesh,
        out_type=jax.ShapeDtypeStruct((B_TOTAL, OUT_DIM), jnp.float32),
        scratch_types=[
            pltpu.VMEM((BPW,), jnp.int32),
            pltpu.VMEM((NBUF, CHUNK, OUT_DIM), jnp.float32),
            pltpu.SemaphoreType.DMA,
            pltpu.SemaphoreType.DMA,
            pltpu.SemaphoreType.DMA,
            pltpu.SemaphoreType.DMA,
        ],
    )
    def gather_kernel(idx_hbm, table_hbm, out_hbm, idx_v, rows_v, s0, s1, s2, s3):
        wid = lax.axis_index("s") * 2 + lax.axis_index("c")
        base = wid * BPW
        sems = (s0, s1, s2, s3)
        pltpu.sync_copy(idx_hbm.at[pl.ds(base, BPW)], idx_v)

        def body(j, carry):
            c0 = j * NBUF
            handles = []
            for b in range(NBUF):
                off = (c0 + b) * CHUNK
                handles.append(
                    pltpu.async_copy(
                        table_hbm.at[idx_v.at[pl.ds(off, CHUNK)]],
                        rows_v.at[b],
                        sems[b],
                    )
                )
            for b in range(NBUF):
                off = (c0 + b) * CHUNK
                handles[b].wait()
                pltpu.sync_copy(rows_v.at[b], out_hbm.at[pl.ds(base + off, CHUNK)])
            return carry

        lax.fori_loop(0, NCHUNK // NBUF, body, 0)

    return gather_kernel


_gather = _make_gather()


def kernel(input_ids, attention_mask, W):
    ids_flat = input_ids.reshape(-1)
    embeds = _gather(ids_flat, W)
    embeds = embeds.reshape(input_ids.shape[0], input_ids.shape[1], OUT_DIM)
    return (embeds, embeds, attention_mask)
